# Initial kernel scaffold; baseline (speedup 1.0000x reference)
import jax
import jax.numpy as jnp
from jax.experimental import pallas as pl


def kernel(x, W):
    raise NotImplementedError("write your pallas kernel here")


# speedup vs baseline: 1.1465x; 1.1465x over previous
import functools

import jax
import jax.numpy as jnp
from jax import lax
from jax.experimental import pallas as pl
from jax.experimental.pallas import tpu as pltpu
from jax.experimental.pallas import tpu_sc as plsc

_ROW_BLOCK = 256
_IDX_CHUNK = 128


def _distance_argmin_body(x_ref, xn_ref, w_ref, en_ref, inds_ref, mind_ref):
    prod = jax.lax.dot_general(
        w_ref[...], x_ref[...],
        dimension_numbers=(((1,), (1,)), ((), ())),
        preferred_element_type=jnp.float32,
    )
    dist = (xn_ref[...] + en_ref[...]) - 2.0 * prod
    mind = jnp.min(dist, axis=0)
    ii = jax.lax.broadcasted_iota(jnp.int32, dist.shape, 0)
    cand = jnp.where(dist == mind[None, :], ii, jnp.int32(dist.shape[0]))
    inds_ref[0, 0, :] = jnp.min(cand, axis=0)
    mind_ref[0, 0, :] = mind


def _sc_gather(Wp, inds):
    V, Dp = Wp.shape
    B = inds.shape[0]
    info = plsc.get_sparse_core_info()
    nw = info.num_cores * info.num_subcores
    b_per_w = B // nw
    chunks = b_per_w // _IDX_CHUNK
    idx2d = inds.reshape(B // _IDX_CHUNK, _IDX_CHUNK)
    mesh = plsc.VectorSubcoreMesh(core_axis_name="c", subcore_axis_name="s")

    @functools.partial(
        pl.kernel,
        mesh=mesh,
        out_type=jax.ShapeDtypeStruct((B, Dp), jnp.float32),
        scratch_types=[
            pltpu.VMEM((chunks, _IDX_CHUNK), jnp.int32),
            pltpu.VMEM((b_per_w, Dp), jnp.float32),
            pltpu.SemaphoreType.DMA,
        ],
    )
    def gather_kernel(table_hbm, idx_hbm, out_hbm, idx_v, rows_v, sem):
        wid = lax.axis_index("s") * info.num_cores + lax.axis_index("c")
        base = wid * b_per_w
        pltpu.sync_copy(idx_hbm.at[pl.ds(wid * chunks, chunks), :], idx_v)
        copies = [
            pltpu.async_copy(
                table_hbm.at[idx_v.at[j]],
                rows_v.at[pl.ds(j * _IDX_CHUNK, _IDX_CHUNK)],
                sem,
            )
            for j in range(chunks)
        ]
        for c in copies:
            c.wait()
        pltpu.sync_copy(rows_v, out_hbm.at[pl.ds(base, b_per_w)])

    return gather_kernel(Wp, idx2d)


def kernel(x, W):
    N, D = x.shape
    V = W.shape[0]
    xn = jnp.sum(x**2, axis=1, keepdims=True)
    en = jnp.sum(W**2, axis=1)
    grid = N // _ROW_BLOCK

    inds3, mind3 = pl.pallas_call(
        _distance_argmin_body,
        grid=(grid,),
        in_specs=[
            pl.BlockSpec((_ROW_BLOCK, D), lambda i: (i, 0)),
            pl.BlockSpec((1, _ROW_BLOCK), lambda i: (0, i)),
            pl.BlockSpec((V, D), lambda i: (0, 0)),
            pl.BlockSpec((V, 1), lambda i: (0, 0)),
        ],
        out_specs=[
            pl.BlockSpec((1, 1, _ROW_BLOCK), lambda i: (i, 0, 0)),
            pl.BlockSpec((1, 1, _ROW_BLOCK), lambda i: (i, 0, 0)),
        ],
        out_shape=[
            jax.ShapeDtypeStruct((grid, 1, _ROW_BLOCK), jnp.int32),
            jax.ShapeDtypeStruct((grid, 1, _ROW_BLOCK), jnp.float32),
        ],
    )(x, xn.reshape(1, N), W, en.reshape(V, 1))

    embed_inds = inds3.reshape(N)
    mean_sq = jnp.sum(mind3) / (N * D)
    loss = mean_sq + mean_sq * 0.25
    Wp = jnp.pad(W, ((0, 0), (0, 128 - D)))
    x_quantized = _sc_gather(Wp, embed_inds)[:, :D]
    return (x_quantized, loss, embed_inds)


# speedup vs baseline: 1.2851x; 1.1209x over previous
import functools

import jax
import jax.numpy as jnp
from jax import lax
from jax.experimental import pallas as pl
from jax.experimental.pallas import tpu as pltpu
from jax.experimental.pallas import tpu_sc as plsc

_ROW_BLOCK = 512
_IDX_CHUNK = 128


def _distance_argmin_body(x_ref, xn_ref, w_ref, en_ref, inds_ref, mind_ref):
    prod = jax.lax.dot_general(
        w_ref[...], x_ref[...],
        dimension_numbers=(((1,), (1,)), ((), ())),
        preferred_element_type=jnp.float32,
    )
    dist = en_ref[...] - 2.0 * prod
    mind = jnp.min(dist, axis=0)
    ii = jax.lax.broadcasted_iota(jnp.int32, dist.shape, 0)
    cand = jnp.where(dist == mind[None, :], ii, jnp.int32(dist.shape[0]))
    inds_ref[0, 0, :] = jnp.min(cand, axis=0)
    mind_ref[0, 0, :] = xn_ref[0, :] + mind


def _sc_gather(Wp, inds):
    V, Dp = Wp.shape
    B = inds.shape[0]
    info = plsc.get_sparse_core_info()
    nw = info.num_cores * info.num_subcores
    b_per_w = B // nw
    chunks = b_per_w // _IDX_CHUNK
    idx2d = inds.reshape(B // _IDX_CHUNK, _IDX_CHUNK)
    mesh = plsc.VectorSubcoreMesh(core_axis_name="c", subcore_axis_name="s")

    @functools.partial(
        pl.kernel,
        mesh=mesh,
        out_type=jax.ShapeDtypeStruct((B, Dp), jnp.float32),
        scratch_types=[
            pltpu.VMEM((chunks, _IDX_CHUNK), jnp.int32),
            pltpu.VMEM((b_per_w, Dp), jnp.float32),
            pltpu.SemaphoreType.DMA,
        ],
    )
    def gather_kernel(table_hbm, idx_hbm, out_hbm, idx_v, rows_v, sem):
        wid = lax.axis_index("s") * info.num_cores + lax.axis_index("c")
        base = wid * b_per_w
        pltpu.sync_copy(idx_hbm.at[pl.ds(wid * chunks, chunks), :], idx_v)
        copies = [
            pltpu.async_copy(
                table_hbm.at[idx_v.at[j]],
                rows_v.at[pl.ds(j * _IDX_CHUNK, _IDX_CHUNK)],
                sem,
            )
            for j in range(chunks)
        ]
        for c in copies:
            c.wait()
        pltpu.sync_copy(rows_v, out_hbm.at[pl.ds(base, b_per_w)])

    return gather_kernel(Wp, idx2d)


def kernel(x, W):
    N, D = x.shape
    V = W.shape[0]
    xn = jnp.sum(x**2, axis=1, keepdims=True)
    en = jnp.sum(W**2, axis=1)
    grid = N // _ROW_BLOCK

    inds3, mind3 = pl.pallas_call(
        _distance_argmin_body,
        grid=(grid,),
        in_specs=[
            pl.BlockSpec((_ROW_BLOCK, D), lambda i: (i, 0)),
            pl.BlockSpec((1, _ROW_BLOCK), lambda i: (0, i)),
            pl.BlockSpec((V, D), lambda i: (0, 0)),
            pl.BlockSpec((V, 1), lambda i: (0, 0)),
        ],
        out_specs=[
            pl.BlockSpec((1, 1, _ROW_BLOCK), lambda i: (i, 0, 0)),
            pl.BlockSpec((1, 1, _ROW_BLOCK), lambda i: (i, 0, 0)),
        ],
        out_shape=[
            jax.ShapeDtypeStruct((grid, 1, _ROW_BLOCK), jnp.int32),
            jax.ShapeDtypeStruct((grid, 1, _ROW_BLOCK), jnp.float32),
        ],
    )(x, xn.reshape(1, N), W, en.reshape(V, 1))

    embed_inds = inds3.reshape(N)
    mean_sq = jnp.sum(mind3) / (N * D)
    loss = mean_sq + mean_sq * 0.25
    Wp = jnp.pad(W, ((0, 0), (0, 128 - D)))
    x_quantized = _sc_gather(Wp, embed_inds)[:, :D]
    return (x_quantized, loss, embed_inds)


# speedup vs baseline: 1.3280x; 1.0334x over previous
import functools

import jax
import jax.numpy as jnp
from jax import lax
from jax.experimental import pallas as pl
from jax.experimental.pallas import tpu as pltpu
from jax.experimental.pallas import tpu_sc as plsc

_ROW_BLOCK = 1024
_IDX_CHUNK = 128


def _distance_argmin_body(x_ref, xn_ref, w_ref, en_ref, inds_ref, mind_ref):
    prod = jax.lax.dot_general(
        w_ref[...], x_ref[...],
        dimension_numbers=(((1,), (1,)), ((), ())),
        preferred_element_type=jnp.float32,
    )
    dist = en_ref[...] - 2.0 * prod
    mind = jnp.min(dist, axis=0)
    ii = jax.lax.broadcasted_iota(jnp.int32, dist.shape, 0)
    cand = jnp.where(dist == mind[None, :], ii, jnp.int32(dist.shape[0]))
    inds_ref[0, 0, :] = jnp.min(cand, axis=0)
    mind_ref[0, 0, :] = xn_ref[0, :] + mind


def _sc_gather(Wp, inds):
    V, Dp = Wp.shape
    B = inds.shape[0]
    info = plsc.get_sparse_core_info()
    nw = info.num_cores * info.num_subcores
    b_per_w = B // nw
    chunks = b_per_w // _IDX_CHUNK
    idx2d = inds.reshape(B // _IDX_CHUNK, _IDX_CHUNK)
    mesh = plsc.VectorSubcoreMesh(core_axis_name="c", subcore_axis_name="s")

    @functools.partial(
        pl.kernel,
        mesh=mesh,
        out_type=jax.ShapeDtypeStruct((B, Dp), jnp.float32),
        scratch_types=[
            pltpu.VMEM((chunks, _IDX_CHUNK), jnp.int32),
            pltpu.VMEM((b_per_w, Dp), jnp.float32),
            pltpu.SemaphoreType.DMA,
        ],
    )
    def gather_kernel(table_hbm, idx_hbm, out_hbm, idx_v, rows_v, sem):
        wid = lax.axis_index("s") * info.num_cores + lax.axis_index("c")
        base = wid * b_per_w
        pltpu.sync_copy(idx_hbm.at[pl.ds(wid * chunks, chunks), :], idx_v)
        copies = [
            pltpu.async_copy(
                table_hbm.at[idx_v.at[j]],
                rows_v.at[pl.ds(j * _IDX_CHUNK, _IDX_CHUNK)],
                sem,
            )
            for j in range(chunks)
        ]
        for c in copies:
            c.wait()
        pltpu.sync_copy(rows_v, out_hbm.at[pl.ds(base, b_per_w)])

    return gather_kernel(Wp, idx2d)


def kernel(x, W):
    N, D = x.shape
    V = W.shape[0]
    xn = jnp.sum(x**2, axis=1, keepdims=True)
    en = jnp.sum(W**2, axis=1)
    grid = N // _ROW_BLOCK

    inds3, mind3 = pl.pallas_call(
        _distance_argmin_body,
        grid=(grid,),
        in_specs=[
            pl.BlockSpec((_ROW_BLOCK, D), lambda i: (i, 0)),
            pl.BlockSpec((1, _ROW_BLOCK), lambda i: (0, i)),
            pl.BlockSpec((V, D), lambda i: (0, 0)),
            pl.BlockSpec((V, 1), lambda i: (0, 0)),
        ],
        out_specs=[
            pl.BlockSpec((1, 1, _ROW_BLOCK), lambda i: (i, 0, 0)),
            pl.BlockSpec((1, 1, _ROW_BLOCK), lambda i: (i, 0, 0)),
        ],
        out_shape=[
            jax.ShapeDtypeStruct((grid, 1, _ROW_BLOCK), jnp.int32),
            jax.ShapeDtypeStruct((grid, 1, _ROW_BLOCK), jnp.float32),
        ],
    )(x, xn.reshape(1, N), W, en.reshape(V, 1))

    embed_inds = inds3.reshape(N)
    mean_sq = jnp.sum(mind3) / (N * D)
    loss = mean_sq + mean_sq * 0.25
    Wp = jnp.pad(W, ((0, 0), (0, 128 - D)))
    x_quantized = _sc_gather(Wp, embed_inds)[:, :D]
    return (x_quantized, loss, embed_inds)
